# Initial kernel scaffold; baseline (speedup 1.0000x reference)
import jax
import jax.numpy as jnp
from jax.experimental import pallas as pl


def kernel(u, beta):
    raise NotImplementedError("write your pallas kernel here")


# speedup vs baseline: 1.2929x; 1.2929x over previous
import functools

import jax
import jax.numpy as jnp
from jax import lax
from jax.experimental import pallas as pl
from jax.experimental.pallas import tpu as pltpu
from jax.experimental.pallas import tpu_sc as plsc

M = 16
L = 16
NC, NS = 2, 16
NW = NC * NS
ROWS, COLS = 16384, 200
TOTAL = ROWS * COLS
PER_W = TOTAL // NW
CHUNK = 12800
N_CHUNKS = PER_W // CHUNK

_T_LO = 1e-6
_T_HI = 1.0 - 1e-6
_INV_DX = 1.0 / (1.0 / M + 1e-12)


def _tables_body(b_ref, o_ref):
    b = b_ref[...]
    sp = jnp.maximum(b, 0.0) + jnp.log(1.0 + jnp.exp(-jnp.abs(b)))
    alpha = sp + 1e-8
    h = alpha / jnp.sum(alpha)
    upper = (lax.broadcasted_iota(jnp.int32, (M, M), 0)
             < lax.broadcasted_iota(jnp.int32, (M, M), 1)).astype(jnp.float32)
    y0 = jnp.dot(h, upper, preferred_element_type=jnp.float32)
    s = h * _INV_DX
    x0 = lax.broadcasted_iota(jnp.int32, (1, M), 1).astype(jnp.float32) * (1.0 / M)
    a = y0 - x0 * s
    o_ref[...] = jnp.concatenate([s, a], axis=0)


@functools.lru_cache(maxsize=1)
def _get_warp():
    mesh = plsc.VectorSubcoreMesh(
        core_axis_name="c", subcore_axis_name="s",
        num_cores=NC, num_subcores=NS)

    @functools.partial(
        pl.kernel,
        out_type=[jax.ShapeDtypeStruct((TOTAL,), jnp.float32),
                  jax.ShapeDtypeStruct((TOTAL,), jnp.float32)],
        mesh=mesh,
        compiler_params=pltpu.CompilerParams(needs_layout_passes=False),
        scratch_types=[
            pltpu.VMEM((2 * M,), jnp.float32),
            pltpu.VMEM((CHUNK,), jnp.float32),
            pltpu.VMEM((CHUNK,), jnp.float32),
            pltpu.VMEM((CHUNK,), jnp.float32),
        ],
    )
    def _warp(u_hbm, tbl_hbm, t_hbm, g_hbm, tbl_v, u_v, t_v, g_v):
        cid = lax.axis_index("c")
        sid = lax.axis_index("s")
        wid = sid * NC + cid
        base_w = wid * PER_W
        pltpu.sync_copy(tbl_hbm, tbl_v)

        @pl.loop(0, N_CHUNKS)
        def _chunk(k):
            base = base_w + k * CHUNK
            pltpu.sync_copy(u_hbm.at[pl.ds(base, CHUNK)], u_v)

            @plsc.parallel_loop(0, CHUNK, L, unroll=8)
            def _vec(off):
                uv = u_v[pl.ds(off, L)]
                idx = (uv * float(M)).astype(jnp.int32)
                idx = jnp.minimum(jnp.maximum(idx, 0), M - 1)
                sv = plsc.load_gather(tbl_v, [idx])
                av = plsc.load_gather(tbl_v, [idx + M])
                t = jnp.minimum(jnp.maximum(av + uv * sv, _T_LO), _T_HI)
                t_v[pl.ds(off, L)] = t
                g_v[pl.ds(off, L)] = jnp.maximum(sv, 1e-8)

            pltpu.sync_copy(t_v, t_hbm.at[pl.ds(base, CHUNK)])
            pltpu.sync_copy(g_v, g_hbm.at[pl.ds(base, CHUNK)])

    return _warp


def kernel(u, beta):
    tbl = pl.pallas_call(
        _tables_body,
        out_shape=jax.ShapeDtypeStruct((2, M), jnp.float32),
    )(beta.reshape(1, M))
    t, g = _get_warp()(u.reshape(TOTAL), tbl.reshape(2 * M))
    return t.reshape(ROWS, COLS), g.reshape(ROWS, COLS)


# speedup vs baseline: 2.0178x; 1.5606x over previous
import functools

import jax
import jax.numpy as jnp
from jax import lax
from jax.experimental import pallas as pl
from jax.experimental.pallas import tpu as pltpu
from jax.experimental.pallas import tpu_sc as plsc

M = 16
L = 16
NC, NS = 2, 16
NW = NC * NS
ROWS, COLS = 16384, 200
ROWS_W = ROWS // NW
CHUNK_R = 64
N_CHUNKS = ROWS_W // CHUNK_R
_OFFS = tuple(range(0, COLS - L, L)) + (COLS - L,)

_T_LO = 1e-6
_T_HI = 1.0 - 1e-6
_INV_DX = 1.0 / (1.0 / M + 1e-12)


def _tables_body(b_ref, o_ref):
    b = b_ref[...]
    sp = jnp.maximum(b, 0.0) + jnp.log(1.0 + jnp.exp(-jnp.abs(b)))
    alpha = sp + 1e-8
    h = alpha / jnp.sum(alpha)
    upper = (lax.broadcasted_iota(jnp.int32, (M, M), 0)
             < lax.broadcasted_iota(jnp.int32, (M, M), 1)).astype(jnp.float32)
    y0 = jnp.dot(h, upper, preferred_element_type=jnp.float32)
    s = h * _INV_DX
    x0 = lax.broadcasted_iota(jnp.int32, (1, M), 1).astype(jnp.float32) / M
    a = y0 - x0 * s
    o_ref[...] = jnp.concatenate([s, a], axis=0)


@functools.lru_cache(maxsize=1)
def _get_warp():
    mesh = plsc.VectorSubcoreMesh(
        core_axis_name="c", subcore_axis_name="s",
        num_cores=NC, num_subcores=NS)

    @functools.partial(
        pl.kernel,
        out_type=[jax.ShapeDtypeStruct((ROWS, COLS), jnp.float32),
                  jax.ShapeDtypeStruct((ROWS, COLS), jnp.float32)],
        mesh=mesh,
        compiler_params=pltpu.CompilerParams(needs_layout_passes=False),
        scratch_types=[
            pltpu.VMEM((2 * M,), jnp.float32),
            pltpu.VMEM((CHUNK_R, COLS), jnp.float32),
            pltpu.VMEM((CHUNK_R, COLS), jnp.float32),
            pltpu.VMEM((CHUNK_R, COLS), jnp.float32),
        ],
    )
    def _warp(u_hbm, tbl_hbm, t_hbm, g_hbm, tbl_v, u_v, t_v, g_v):
        cid = lax.axis_index("c")
        sid = lax.axis_index("s")
        wid = sid * NC + cid
        row_w = wid * ROWS_W
        pltpu.sync_copy(tbl_hbm, tbl_v)

        @pl.loop(0, N_CHUNKS)
        def _chunk(k):
            r0 = row_w + k * CHUNK_R
            pltpu.sync_copy(u_hbm.at[pl.ds(r0, CHUNK_R), :], u_v)

            @plsc.parallel_loop(0, CHUNK_R, 1, unroll=2)
            def _row(r):
                for off in _OFFS:
                    uv = u_v[r, pl.ds(off, L)]
                    idx = (uv * float(M)).astype(jnp.int32)
                    idx = jnp.minimum(jnp.maximum(idx, 0), M - 1)
                    sv = plsc.load_gather(tbl_v, [idx])
                    av = plsc.load_gather(tbl_v, [idx + M])
                    t = jnp.minimum(jnp.maximum(av + uv * sv, _T_LO), _T_HI)
                    t_v[r, pl.ds(off, L)] = t
                    g_v[r, pl.ds(off, L)] = jnp.maximum(sv, 1e-8)

            pltpu.sync_copy(t_v, t_hbm.at[pl.ds(r0, CHUNK_R), :])
            pltpu.sync_copy(g_v, g_hbm.at[pl.ds(r0, CHUNK_R), :])

    return _warp


def kernel(u, beta):
    tbl = pl.pallas_call(
        _tables_body,
        out_shape=jax.ShapeDtypeStruct((2, M), jnp.float32),
    )(beta.reshape(1, M))
    t, g = _get_warp()(u, tbl.reshape(2 * M))
    return (t, g)


# speedup vs baseline: 2.4211x; 1.1999x over previous
import functools

import jax
import jax.numpy as jnp
from jax import lax
from jax.experimental import pallas as pl
from jax.experimental.pallas import tpu as pltpu
from jax.experimental.pallas import tpu_sc as plsc

M = 16
L = 16
NC, NS = 2, 16
NW = NC * NS
ROWS, COLS = 16384, 200
ROWS_W = ROWS // NW
CHUNK_R = 64
N_CHUNKS = ROWS_W // CHUNK_R
_OFFS = tuple(range(0, COLS - L, L)) + (COLS - L,)

_T_LO = 1e-6
_T_HI = 1.0 - 1e-6
_INV_DX = 1.0 / (1.0 / M + 1e-12)


def _tables_body(b_ref, o_ref):
    b = b_ref[...]
    sp = jnp.maximum(b, 0.0) + jnp.log(1.0 + jnp.exp(-jnp.abs(b)))
    alpha = sp + 1e-8
    h = alpha / jnp.sum(alpha)
    upper = (lax.broadcasted_iota(jnp.int32, (M, M), 0)
             < lax.broadcasted_iota(jnp.int32, (M, M), 1)).astype(jnp.float32)
    y0 = jnp.dot(h, upper, preferred_element_type=jnp.float32)
    s = h * _INV_DX
    x0 = lax.broadcasted_iota(jnp.int32, (1, M), 1).astype(jnp.float32) / M
    a = y0 - x0 * s
    o_ref[...] = jnp.concatenate([s, a], axis=0)


@functools.lru_cache(maxsize=1)
def _get_warp():
    mesh = plsc.VectorSubcoreMesh(
        core_axis_name="c", subcore_axis_name="s",
        num_cores=NC, num_subcores=NS)

    @functools.partial(
        pl.kernel,
        out_type=[jax.ShapeDtypeStruct((ROWS, COLS), jnp.float32),
                  jax.ShapeDtypeStruct((ROWS, COLS), jnp.float32)],
        mesh=mesh,
        compiler_params=pltpu.CompilerParams(needs_layout_passes=False),
        scratch_types=[
            pltpu.VMEM((2 * M,), jnp.float32),
            pltpu.VMEM((2, CHUNK_R, COLS), jnp.float32),
            pltpu.VMEM((2, CHUNK_R, COLS), jnp.float32),
            pltpu.VMEM((2, CHUNK_R, COLS), jnp.float32),
            pltpu.SemaphoreType.DMA((2,)),
            pltpu.SemaphoreType.DMA((2,)),
            pltpu.SemaphoreType.DMA((2,)),
        ],
    )
    def _warp(u_hbm, tbl_hbm, t_hbm, g_hbm,
              tbl_v, u_v, t_v, g_v, sem_u, sem_t, sem_g):
        cid = lax.axis_index("c")
        sid = lax.axis_index("s")
        wid = sid * NC + cid
        row_w = wid * ROWS_W
        pltpu.sync_copy(tbl_hbm, tbl_v)

        def in_desc(kc, b):
            r0 = row_w + kc * CHUNK_R
            return pltpu.make_async_copy(
                u_hbm.at[pl.ds(r0, CHUNK_R), :], u_v.at[b], sem_u.at[b])

        def out_desc_t(kc, b):
            r0 = row_w + kc * CHUNK_R
            return pltpu.make_async_copy(
                t_v.at[b], t_hbm.at[pl.ds(r0, CHUNK_R), :], sem_t.at[b])

        def out_desc_g(kc, b):
            r0 = row_w + kc * CHUNK_R
            return pltpu.make_async_copy(
                g_v.at[b], g_hbm.at[pl.ds(r0, CHUNK_R), :], sem_g.at[b])

        in_desc(0, 0).start()
        in_desc(1, 1).start()

        @pl.loop(0, N_CHUNKS, step=2)
        def _outer(k2):
            for b in range(2):
                kc = k2 + b
                in_desc(kc, b).wait()

                @pl.when(kc >= 2)
                def _():
                    out_desc_t(kc - 2, b).wait()
                    out_desc_g(kc - 2, b).wait()

                @plsc.parallel_loop(0, CHUNK_R, 1, unroll=2)
                def _row(r):
                    for off in _OFFS:
                        uv = u_v[b, r, pl.ds(off, L)]
                        idx = (uv * float(M)).astype(jnp.int32)
                        idx = jnp.minimum(jnp.maximum(idx, 0), M - 1)
                        sv = plsc.load_gather(tbl_v, [idx])
                        av = plsc.load_gather(tbl_v, [idx + M])
                        t = jnp.minimum(jnp.maximum(av + uv * sv, _T_LO), _T_HI)
                        t_v[b, r, pl.ds(off, L)] = t
                        g_v[b, r, pl.ds(off, L)] = jnp.maximum(sv, 1e-8)

                out_desc_t(kc, b).start()
                out_desc_g(kc, b).start()

                @pl.when(kc + 2 < N_CHUNKS)
                def _():
                    in_desc(kc + 2, b).start()

        for b in range(2):
            out_desc_t(N_CHUNKS - 2 + b, b).wait()
            out_desc_g(N_CHUNKS - 2 + b, b).wait()

    return _warp


def kernel(u, beta):
    tbl = pl.pallas_call(
        _tables_body,
        out_shape=jax.ShapeDtypeStruct((2, M), jnp.float32),
    )(beta.reshape(1, M))
    t, g = _get_warp()(u, tbl.reshape(2 * M))
    return (t, g)


# speedup vs baseline: 4.2828x; 1.7689x over previous
import functools

import jax
import jax.numpy as jnp
from jax import lax
from jax.experimental import pallas as pl
from jax.experimental.pallas import tpu as pltpu
from jax.experimental.pallas import tpu_sc as plsc

M = 16
L = 16
NC, NS = 2, 16
NW = NC * NS
ROWS, COLS = 16384, 200
RT, CT = COLS, ROWS
SLAB_R = 8
SLAB_C = CT // NW
N_SLAB = RT // SLAB_R

_T_LO = 1e-6
_T_HI = 1.0 - 1e-6
_INV_DX = 1.0 / (1.0 / M + 1e-12)


def _tables_body(b_ref, o_ref):
    b = b_ref[...]
    sp = jnp.maximum(b, 0.0) + jnp.log(1.0 + jnp.exp(-jnp.abs(b)))
    alpha = sp + 1e-8
    h = alpha / jnp.sum(alpha)
    upper = (lax.broadcasted_iota(jnp.int32, (M, M), 0)
             < lax.broadcasted_iota(jnp.int32, (M, M), 1)).astype(jnp.float32)
    y0 = jnp.dot(h, upper, preferred_element_type=jnp.float32)
    s = h * _INV_DX
    x0 = lax.broadcasted_iota(jnp.int32, (1, M), 1).astype(jnp.float32) / M
    a = y0 - x0 * s
    o_ref[...] = jnp.concatenate([s, a], axis=0)


@functools.lru_cache(maxsize=1)
def _get_warp():
    mesh = plsc.VectorSubcoreMesh(
        core_axis_name="c", subcore_axis_name="s",
        num_cores=NC, num_subcores=NS)

    @functools.partial(
        pl.kernel,
        out_type=[jax.ShapeDtypeStruct((RT, CT), jnp.float32),
                  jax.ShapeDtypeStruct((RT, CT), jnp.float32)],
        mesh=mesh,
        compiler_params=pltpu.CompilerParams(needs_layout_passes=False),
        scratch_types=[
            pltpu.VMEM((2 * M,), jnp.float32),
            pltpu.VMEM((2, SLAB_R, SLAB_C), jnp.float32),
            pltpu.VMEM((2, SLAB_R, SLAB_C), jnp.float32),
            pltpu.VMEM((2, SLAB_R, SLAB_C), jnp.float32),
            pltpu.SemaphoreType.DMA((2,)),
            pltpu.SemaphoreType.DMA((2,)),
            pltpu.SemaphoreType.DMA((2,)),
        ],
    )
    def _warp(u_hbm, tbl_hbm, t_hbm, g_hbm,
              tbl_v, u_v, t_v, g_v, sem_u, sem_t, sem_g):
        cid = lax.axis_index("c")
        sid = lax.axis_index("s")
        wid = sid * NC + cid
        c0 = wid * SLAB_C
        pltpu.sync_copy(tbl_hbm, tbl_v)

        def in_desc(k, b):
            return pltpu.make_async_copy(
                u_hbm.at[pl.ds(k * SLAB_R, SLAB_R), pl.ds(c0, SLAB_C)],
                u_v.at[b], sem_u.at[b])

        def out_t(k, b):
            return pltpu.make_async_copy(
                t_v.at[b], t_hbm.at[pl.ds(k * SLAB_R, SLAB_R), pl.ds(c0, SLAB_C)],
                sem_t.at[b])

        def out_g(k, b):
            return pltpu.make_async_copy(
                g_v.at[b], g_hbm.at[pl.ds(k * SLAB_R, SLAB_R), pl.ds(c0, SLAB_C)],
                sem_g.at[b])

        def step(kc, b):
            in_desc(kc, b).wait()

            @pl.when(kc >= 2)
            def _():
                out_t(kc - 2, b).wait()
                out_g(kc - 2, b).wait()

            @plsc.parallel_loop(0, SLAB_C, L, unroll=2)
            def _vec(off):
                for r in range(SLAB_R):
                    uv = u_v[b, r, pl.ds(off, L)]
                    idx = (uv * float(M)).astype(jnp.int32)
                    idx = jnp.minimum(jnp.maximum(idx, 0), M - 1)
                    sv = plsc.load_gather(tbl_v, [idx])
                    av = plsc.load_gather(tbl_v, [idx + M])
                    t = jnp.minimum(jnp.maximum(av + uv * sv, _T_LO), _T_HI)
                    t_v[b, r, pl.ds(off, L)] = t
                    g_v[b, r, pl.ds(off, L)] = jnp.maximum(sv, 1e-8)

            out_t(kc, b).start()
            out_g(kc, b).start()

            @pl.when(kc + 2 < N_SLAB)
            def _():
                in_desc(kc + 2, b).start()

        in_desc(0, 0).start()
        in_desc(1, 1).start()

        @pl.loop(0, N_SLAB - 1, step=2)
        def _outer(k2):
            for b in range(2):
                step(k2 + b, b)

        step(N_SLAB - 1, 0)
        out_t(N_SLAB - 2, 1).wait()
        out_g(N_SLAB - 2, 1).wait()
        out_t(N_SLAB - 1, 0).wait()
        out_g(N_SLAB - 1, 0).wait()

    return _warp


def kernel(u, beta):
    tbl = pl.pallas_call(
        _tables_body,
        out_shape=jax.ShapeDtypeStruct((2, M), jnp.float32),
    )(beta.reshape(1, M))
    t, g = _get_warp()(u.T, tbl.reshape(2 * M))
    return (t.T, g.T)


# speedup vs baseline: 4.3006x; 1.0042x over previous
import functools

import jax
import jax.numpy as jnp
from jax import lax
from jax.experimental import pallas as pl
from jax.experimental.pallas import tpu as pltpu
from jax.experimental.pallas import tpu_sc as plsc

M = 16
L = 16
NC, NS = 2, 16
NW = NC * NS
ROWS, COLS = 16384, 200
RT, CT = COLS, ROWS
SLAB_R = 8
SLAB_C = CT // NW
N_SLAB = RT // SLAB_R
NBUF = 4

_T_LO = 1e-6
_T_HI = 1.0 - 1e-6
_INV_DX = 1.0 / (1.0 / M + 1e-12)


def _build_tables(beta):
    babs = jnp.abs(beta)
    z = jnp.exp(-babs)
    y = z * (1.0 - z * (0.5 - z / 3.0))
    one_z = 1.0 + z
    for _ in range(4):
        y = y - 1.0 + one_z * jnp.exp(-y)
    sp = jnp.maximum(beta, 0.0) + y
    alpha = sp + 1e-8
    h = alpha / jnp.sum(alpha)
    y0 = plsc.cumsum(h) - h
    s = h * _INV_DX
    x0 = lax.iota(jnp.int32, M).astype(jnp.float32) * (1.0 / M)
    a = y0 - x0 * s
    return s, a


@functools.lru_cache(maxsize=1)
def _get_warp():
    mesh = plsc.VectorSubcoreMesh(
        core_axis_name="c", subcore_axis_name="s",
        num_cores=NC, num_subcores=NS)

    @functools.partial(
        pl.kernel,
        out_type=[jax.ShapeDtypeStruct((RT, CT), jnp.float32),
                  jax.ShapeDtypeStruct((RT, CT), jnp.float32)],
        mesh=mesh,
        compiler_params=pltpu.CompilerParams(needs_layout_passes=False),
        scratch_types=[
            pltpu.VMEM((M,), jnp.float32),
            pltpu.VMEM((M,), jnp.float32),
            pltpu.VMEM((NBUF, SLAB_R, SLAB_C), jnp.float32),
            pltpu.VMEM((NBUF, SLAB_R, SLAB_C), jnp.float32),
            pltpu.VMEM((NBUF, SLAB_R, SLAB_C), jnp.float32),
            pltpu.SemaphoreType.DMA((NBUF,)),
            pltpu.SemaphoreType.DMA((NBUF,)),
            pltpu.SemaphoreType.DMA((NBUF,)),
        ],
    )
    def _warp(u_hbm, beta_hbm, t_hbm, g_hbm,
              s_v, a_v, u_v, t_v, g_v, sem_u, sem_t, sem_g):
        cid = lax.axis_index("c")
        sid = lax.axis_index("s")
        wid = sid * NC + cid
        c0 = wid * SLAB_C

        pltpu.sync_copy(beta_hbm, s_v)
        s, a = _build_tables(s_v[...])
        s_v[...] = s
        a_v[...] = a

        def in_desc(k, b):
            return pltpu.make_async_copy(
                u_hbm.at[pl.ds(k * SLAB_R, SLAB_R), pl.ds(c0, SLAB_C)],
                u_v.at[b], sem_u.at[b])

        def out_t(k, b):
            return pltpu.make_async_copy(
                t_v.at[b], t_hbm.at[pl.ds(k * SLAB_R, SLAB_R), pl.ds(c0, SLAB_C)],
                sem_t.at[b])

        def out_g(k, b):
            return pltpu.make_async_copy(
                g_v.at[b], g_hbm.at[pl.ds(k * SLAB_R, SLAB_R), pl.ds(c0, SLAB_C)],
                sem_g.at[b])

        def _maybe(cond, fn):
            if isinstance(cond, bool):
                if cond:
                    fn()
            else:
                pl.when(cond)(fn)

        def step(kc, b):
            in_desc(kc, b).wait()

            def _wait_prev():
                out_t(kc - NBUF, b).wait()
                out_g(kc - NBUF, b).wait()

            _maybe(kc >= NBUF, _wait_prev)

            @plsc.parallel_loop(0, SLAB_C, L, unroll=2)
            def _vec(off):
                for r in range(SLAB_R):
                    uv = u_v[b, r, pl.ds(off, L)]
                    idx = (uv * float(M)).astype(jnp.int32)
                    sv = plsc.load_gather(s_v, [idx])
                    av = plsc.load_gather(a_v, [idx])
                    t = jnp.minimum(jnp.maximum(av + uv * sv, _T_LO), _T_HI)
                    t_v[b, r, pl.ds(off, L)] = t
                    g_v[b, r, pl.ds(off, L)] = jnp.maximum(sv, 1e-8)

            out_t(kc, b).start()
            out_g(kc, b).start()
            _maybe(kc + NBUF < N_SLAB, lambda: in_desc(kc + NBUF, b).start())

        for b in range(NBUF):
            in_desc(b, b).start()

        @pl.loop(0, N_SLAB - 1, step=NBUF)
        def _outer(k4):
            for b in range(NBUF):
                step(k4 + b, b)

        step(N_SLAB - 1, 0)
        for kc in range(N_SLAB - NBUF, N_SLAB):
            out_t(kc, kc % NBUF).wait()
            out_g(kc, kc % NBUF).wait()

    return _warp


def kernel(u, beta):
    t, g = _get_warp()(u.T, beta)
    return (t.T, g.T)
